# Initial kernel scaffold; baseline (speedup 1.0000x reference)
import jax
import jax.numpy as jnp
from jax.experimental import pallas as pl


def kernel(loc_p, conf_p, land_p, loc_t, conf_t, land_t):
    raise NotImplementedError("write your pallas kernel here")


# speedup vs baseline: 1.5285x; 1.5285x over previous
import functools

import jax
import jax.numpy as jnp
from jax import lax
from jax.experimental import pallas as pl
from jax.experimental.pallas import tpu as pltpu

B, P = 32, 16800
N = B * P
LANES = 128
ROWS = N // LANES
GRID = 21
RB = ROWS // GRID
NEG_POS_RATIO = 7
BOX_WEIGHT = 2.0
INT32_MIN = -2147483648


def _group_sum_mat(group, lanes=LANES):
    j = lax.broadcasted_iota(jnp.int32, (group * lanes, lanes), 0)
    i = lax.broadcasted_iota(jnp.int32, (group * lanes, lanes), 1)
    return (j // group == i).astype(jnp.float32)


def _diff_mat(lanes=LANES):
    j = lax.broadcasted_iota(jnp.int32, (2 * lanes, lanes), 0)
    i = lax.broadcasted_iota(jnp.int32, (2 * lanes, lanes), 1)
    return (j == 2 * i + 1).astype(jnp.float32) - (j == 2 * i).astype(jnp.float32)


def _smooth_l1(x):
    a = jnp.abs(x)
    return jnp.where(a < 1.0, 0.5 * x * x, a - 0.5)


def _softplus(z):
    return jnp.maximum(z, 0.0) + jnp.log(1.0 + jnp.exp(-jnp.abs(z)))


def _mbl_kernel(lp, lt, dp, dt, cp, ct, out, key_s, nl_s, accf, acci):
    step = pl.program_id(0)

    @pl.when(step == 0)
    def _init():
        accf[0] = 0.0
        accf[1] = 0.0
        accf[2] = 0.0
        acci[0] = 0
        acci[1] = 0

    labels = ct[...]
    pos = labels > 0
    neg = labels == 0
    posf = pos.astype(jnp.float32)

    d = lax.dot_general(cp[...], _diff_mat(), (((1,), (0,)), ((), ())),
                        preferred_element_type=jnp.float32)
    z = jnp.where(pos, -d, d)
    spl = _softplus(z)

    accf[0] += jnp.sum(posf * spl)
    acci[0] += jnp.sum(pos, dtype=jnp.int32)
    acci[1] += jnp.sum(neg, dtype=jnp.int32)

    bits = lax.bitcast_convert_type(d, jnp.int32)
    key = jnp.where(bits >= 0, bits, INT32_MIN - bits)
    key = jnp.where(neg, key, INT32_MIN)
    key_s[pl.ds(step * RB, RB), :] = key
    nl_s[pl.ds(step * RB, RB), :] = jnp.where(neg, spl, 0.0)

    s4 = _smooth_l1(lp[...] - lt[...])
    t4 = lax.dot_general(s4, _group_sum_mat(4), (((1,), (0,)), ((), ())),
                         preferred_element_type=jnp.float32)
    accf[1] += jnp.sum(t4 * posf)

    s10 = _smooth_l1(dp[...] - dt[...])
    t10 = lax.dot_general(s10, _group_sum_mat(10), (((1,), (0,)), ((), ())),
                          preferred_element_type=jnp.float32)
    bad = (dt[...] == -1.0).astype(jnp.float32)
    badcnt = lax.dot_general(bad, _group_sum_mat(10), (((1,), (0,)), ((), ())),
                             preferred_element_type=jnp.float32)
    lm_mask = posf * (badcnt < 0.5).astype(jnp.float32)
    accf[2] += jnp.sum(t10 * lm_mask)

    @pl.when(step == GRID - 1)
    def _finalize():
        cnt_pos = acci[0]
        cnt_neg = acci[1]
        k = jnp.minimum(NEG_POS_RATIO * cnt_pos, cnt_neg)

        def count_ge(cand):
            def body(ci, c):
                blk = key_s[pl.ds(ci * RB, RB), :]
                return c + jnp.sum(blk >= cand, dtype=jnp.int32)
            return lax.fori_loop(0, GRID, body, jnp.int32(0))

        t0 = jnp.where(count_ge(jnp.int32(0)) >= k, jnp.int32(0), INT32_MIN)

        def bs_body(i, t):
            stp = jnp.int32(1) << (30 - i)
            cand = t + stp
            return jnp.where(count_ge(cand) >= k, cand, t)

        t = lax.fori_loop(0, 31, bs_body, t0)

        def fin_body(ci, carry):
            cg, sg, ce, se = carry
            kb = key_s[pl.ds(ci * RB, RB), :]
            vb = nl_s[pl.ds(ci * RB, RB), :]
            gt = kb > t
            eq = kb == t
            cg += jnp.sum(gt, dtype=jnp.int32)
            sg += jnp.sum(jnp.where(gt, vb, 0.0))
            ce += jnp.sum(eq, dtype=jnp.int32)
            se += jnp.sum(jnp.where(eq, vb, 0.0))
            return cg, sg, ce, se

        cg, sg, ce, se = lax.fori_loop(
            0, GRID, fin_body,
            (jnp.int32(0), jnp.float32(0.0), jnp.int32(0), jnp.float32(0.0)))

        tval = se / jnp.maximum(ce, 1).astype(jnp.float32)
        sum_topk = jnp.where(k > 0,
                             sg + (k - cg).astype(jnp.float32) * tval,
                             0.0)

        nf = jnp.maximum(1.0, cnt_pos.astype(jnp.float32))
        v0 = (accf[0] + sum_topk) / nf
        v1 = BOX_WEIGHT * accf[1] / nf
        v2 = accf[2] / nf

        r = lax.broadcasted_iota(jnp.int32, (8, LANES), 0)
        c = lax.broadcasted_iota(jnp.int32, (8, LANES), 1)
        outv = jnp.where((r == 0) & (c == 0), v0,
                         jnp.where((r == 0) & (c == 1), v1,
                                   jnp.where((r == 0) & (c == 2), v2, 0.0)))
        out[...] = outv


@jax.jit
def kernel(loc_p, conf_p, land_p, loc_t, conf_t, land_t):
    lp = loc_p.reshape(ROWS, 4 * LANES)
    lt = loc_t.reshape(ROWS, 4 * LANES)
    dp = land_p.reshape(ROWS, 10 * LANES)
    dt = land_t.reshape(ROWS, 10 * LANES)
    cp = conf_p.reshape(ROWS, 2 * LANES)
    ct = conf_t.reshape(ROWS, LANES).astype(jnp.int32)

    out = pl.pallas_call(
        _mbl_kernel,
        grid=(GRID,),
        in_specs=[
            pl.BlockSpec((RB, 4 * LANES), lambda i: (i, 0)),
            pl.BlockSpec((RB, 4 * LANES), lambda i: (i, 0)),
            pl.BlockSpec((RB, 10 * LANES), lambda i: (i, 0)),
            pl.BlockSpec((RB, 10 * LANES), lambda i: (i, 0)),
            pl.BlockSpec((RB, 2 * LANES), lambda i: (i, 0)),
            pl.BlockSpec((RB, LANES), lambda i: (i, 0)),
        ],
        out_specs=pl.BlockSpec((8, LANES), lambda i: (0, 0)),
        out_shape=jax.ShapeDtypeStruct((8, LANES), jnp.float32),
        scratch_shapes=[
            pltpu.VMEM((ROWS, LANES), jnp.int32),
            pltpu.VMEM((ROWS, LANES), jnp.float32),
            pltpu.SMEM((4,), jnp.float32),
            pltpu.SMEM((4,), jnp.int32),
        ],
    )(lp, lt, dp, dt, cp, ct)

    return (out[0, 0], out[0, 1], out[0, 2])


# speedup vs baseline: 39.2402x; 25.6723x over previous
import jax
import jax.numpy as jnp
from jax import lax
from jax.experimental import pallas as pl
from jax.experimental.pallas import tpu as pltpu

B, P = 32, 16800
LANES = 128
GRID = 5
NEG_POS_RATIO = 7
BOX_WEIGHT = 2.0
INT32_MIN = -2147483648


def _smooth_l1(x):
    a = jnp.abs(x)
    return jnp.where(a < 1.0, 0.5 * x * x, a - 0.5)


def _mbl_kernel(ct, cp, lp, lt, dp, dt, out, key_s, nl_s, accf, acci):
    step = pl.program_id(0)

    @pl.when(step == 0)
    def _init():
        accf[0] = 0.0
        accf[1] = 0.0
        accf[2] = 0.0
        acci[0] = 0
        acci[1] = 0

    @pl.when(step < GRID - 1)
    def _accumulate():
        labels = ct[...]
        pos = labels > 0
        neg = labels == 0

        x0 = cp[:, 0, :]
        x1 = cp[:, 1, :]
        d = x1 - x0
        z = jnp.where(pos, -d, d)
        spl = jnp.maximum(z, 0.0) + jnp.log(1.0 + jnp.exp(-jnp.abs(z)))

        accf[0] += jnp.sum(jnp.where(pos, spl, 0.0))
        acci[0] += jnp.sum(pos, dtype=jnp.int32)
        acci[1] += jnp.sum(neg, dtype=jnp.int32)

        bits = lax.bitcast_convert_type(d, jnp.int32)
        key = jnp.where(bits >= 0, bits, INT32_MIN - bits)
        key = jnp.where(neg, key, INT32_MIN)
        key_s[pl.ds(step * 8, 8), :] = key
        nl_s[pl.ds(step * 8, 8), :] = jnp.where(neg, spl, 0.0)

        t4 = _smooth_l1(lp[:, 0, :] - lt[:, 0, :])
        for c in range(1, 4):
            t4 += _smooth_l1(lp[:, c, :] - lt[:, c, :])
        accf[1] += jnp.sum(jnp.where(pos, t4, 0.0))

        t10 = _smooth_l1(dp[0] - dt[0])
        badc = (dt[0] == -1.0).astype(jnp.int32)
        for c in range(1, 10):
            t10 += _smooth_l1(dp[c] - dt[c])
            badc += (dt[c] == -1.0).astype(jnp.int32)
        lm = pos & (badc == 0)
        accf[2] += jnp.sum(jnp.where(lm, t10, 0.0))

    @pl.when(step == GRID - 1)
    def _finalize():
        cnt_pos = acci[0]
        cnt_neg = acci[1]
        k = jnp.minimum(NEG_POS_RATIO * cnt_pos, cnt_neg)

        def count_ge(cand):
            def body(ci, c):
                blk = key_s[pl.ds(ci * 8, 8), :]
                return c + jnp.sum(blk >= cand, dtype=jnp.int32)
            return lax.fori_loop(0, 4, body, jnp.int32(0))

        t0 = jnp.where(count_ge(jnp.int32(0)) >= k, jnp.int32(0),
                       jnp.int32(INT32_MIN))

        def bs_body(i, t):
            stp = jnp.int32(1) << (30 - i)
            cand = t + stp
            return jnp.where(count_ge(cand) >= k, cand, t)

        t = lax.fori_loop(0, 31, bs_body, t0)

        def fin_body(ci, carry):
            cg, sg, ce, se = carry
            kb = key_s[pl.ds(ci * 8, 8), :]
            vb = nl_s[pl.ds(ci * 8, 8), :]
            gt = kb > t
            eq = kb == t
            cg += jnp.sum(gt, dtype=jnp.int32)
            sg += jnp.sum(jnp.where(gt, vb, 0.0))
            ce += jnp.sum(eq, dtype=jnp.int32)
            se += jnp.sum(jnp.where(eq, vb, 0.0))
            return cg, sg, ce, se

        cg, sg, ce, se = lax.fori_loop(
            0, 4, fin_body,
            (jnp.int32(0), jnp.float32(0.0), jnp.int32(0), jnp.float32(0.0)))

        tval = se / jnp.maximum(ce, 1).astype(jnp.float32)
        sum_topk = jnp.where(k > 0,
                             sg + (k - cg).astype(jnp.float32) * tval,
                             0.0)

        nf = jnp.maximum(1.0, cnt_pos.astype(jnp.float32))
        v0 = (accf[0] + sum_topk) / nf
        v1 = BOX_WEIGHT * accf[1] / nf
        v2 = accf[2] / nf

        r = lax.broadcasted_iota(jnp.int32, (8, LANES), 0)
        c = lax.broadcasted_iota(jnp.int32, (8, LANES), 1)
        outv = jnp.where((r == 0) & (c == 0), v0,
                         jnp.where((r == 0) & (c == 1), v1,
                                   jnp.where((r == 0) & (c == 2), v2, 0.0)))
        out[...] = outv


@jax.jit
def kernel(loc_p, conf_p, land_p, loc_t, conf_t, land_t):
    ct = conf_t.astype(jnp.int32)
    cpv = conf_p.transpose(0, 2, 1)
    lpv = loc_p.transpose(0, 2, 1)
    ltv = loc_t.transpose(0, 2, 1)
    dpv = land_p.transpose(2, 0, 1)
    dtv = land_t.transpose(2, 0, 1)

    bt = lambda s: jnp.minimum(s, GRID - 2)

    out = pl.pallas_call(
        _mbl_kernel,
        grid=(GRID,),
        in_specs=[
            pl.BlockSpec((8, P), lambda s: (bt(s), 0)),
            pl.BlockSpec((8, 2, P), lambda s: (bt(s), 0, 0)),
            pl.BlockSpec((8, 4, P), lambda s: (bt(s), 0, 0)),
            pl.BlockSpec((8, 4, P), lambda s: (bt(s), 0, 0)),
            pl.BlockSpec((10, 8, P), lambda s: (0, bt(s), 0)),
            pl.BlockSpec((10, 8, P), lambda s: (0, bt(s), 0)),
        ],
        out_specs=pl.BlockSpec((8, LANES), lambda s: (0, 0)),
        out_shape=jax.ShapeDtypeStruct((8, LANES), jnp.float32),
        scratch_shapes=[
            pltpu.VMEM((B, P), jnp.int32),
            pltpu.VMEM((B, P), jnp.float32),
            pltpu.SMEM((4,), jnp.float32),
            pltpu.SMEM((4,), jnp.int32),
        ],
    )(ct, cpv, lpv, ltv, dpv, dtv)

    return (out[0, 0], out[0, 1], out[0, 2])
